# baseline (speedup 1.0000x reference)
import functools

import jax
import jax.numpy as jnp
from jax import lax
from jax.experimental import pallas as pl
from jax.experimental.pallas import tpu as pltpu
from jax.experimental.pallas import tpu_sc as plsc

_NC = 2
_NS = 16
_NW = _NC * _NS
_LANES = 16


def _sc_mean_pool(x, table):
    nrows, hist = x.shape
    _, d = table.shape
    rows_per_w = nrows // _NW
    nvec = d // _LANES
    inv = 1.0 / float(hist)

    mesh = plsc.VectorSubcoreMesh(core_axis_name="c", subcore_axis_name="s")
    nbuf = 4
    unroll = 4
    assert hist % unroll == 0 and rows_per_w % nbuf == 0

    @functools.partial(
        pl.kernel,
        mesh=mesh,
        compiler_params=pltpu.CompilerParams(use_tc_tiling_on_sc=False),
        out_type=jax.ShapeDtypeStruct((nrows, d), jnp.float32),
        scratch_types=[
            pltpu.VMEM((rows_per_w, hist), jnp.int32),
            [pltpu.VMEM((hist, d), jnp.float32)] * nbuf,
            pltpu.VMEM((rows_per_w, d), jnp.float32),
            [pltpu.SemaphoreType.DMA] * nbuf,
        ],
    )
    def pool(x_hbm, table_hbm, out_hbm, idx, bufs, pooled_v, sems):
        cid = lax.axis_index("c")
        sid = lax.axis_index("s")
        wid = sid * _NC + cid
        base = wid * rows_per_w
        pltpu.sync_copy(x_hbm.at[pl.ds(base, rows_per_w)], idx)

        def gather_row(r, buf, sem):
            pltpu.async_copy(table_hbm.at[idx.at[r]], buf, sem)

        for k in range(nbuf):
            gather_row(k, bufs[k], sems[k])

        def chunk_sum(buf, acc):
            def body(i, acc):
                r = unroll * i
                out = []
                for k in range(nvec):
                    sl = pl.ds(_LANES * k, _LANES)
                    v0 = buf[r, sl] + buf[r + 1, sl]
                    v1 = buf[r + 2, sl] + buf[r + 3, sl]
                    out.append(acc[k] + (v0 + v1))
                return tuple(out)
            return lax.fori_loop(0, hist // unroll, body, acc)

        zero = jnp.zeros((_LANES,), jnp.float32)

        def outer(j, carry):
            for u in range(nbuf):
                r = nbuf * j + u
                buf = bufs[u]
                pltpu.make_async_copy(
                    table_hbm.at[idx.at[0]], buf, sems[u]).wait()
                acc = chunk_sum(buf, (zero,) * nvec)

                @pl.when(r + nbuf < rows_per_w)
                def _():
                    gather_row(r + nbuf, buf, sems[u])

                for k in range(nvec):
                    pooled_v[r, pl.ds(_LANES * k, _LANES)] = acc[k] * inv
            return carry

        lax.fori_loop(0, rows_per_w // nbuf, outer, 0)
        pltpu.sync_copy(pooled_v, out_hbm.at[pl.ds(base, rows_per_w)])

    return pool(x, table)


def _tc_head(pooled, w, b):
    bn, d = pooled.shape
    n = w.shape[1]

    def body(p_ref, w_ref, b_ref, o_ref):
        z = jnp.dot(p_ref[...], w_ref[...],
                    preferred_element_type=jnp.float32) + b_ref[...]
        m = jnp.max(z, axis=1, keepdims=True)
        e = jnp.exp(z - m)
        o_ref[...] = e / jnp.sum(e, axis=1, keepdims=True)

    return pl.pallas_call(
        body,
        grid=(1,),
        in_specs=[
            pl.BlockSpec((bn, d), lambda i: (0, 0)),
            pl.BlockSpec((d, n), lambda i: (0, 0)),
            pl.BlockSpec((1, n), lambda i: (0, 0)),
        ],
        out_specs=pl.BlockSpec((bn, n), lambda i: (0, 0)),
        out_shape=jax.ShapeDtypeStruct((bn, n), jnp.float32),
    )(pooled, w, b.reshape(1, n))


def kernel(x, table, W, b):
    pooled = _sc_mean_pool(x, table)
    return _tc_head(pooled, W, b)
